# baseline (speedup 1.0000x reference)
import dataclasses
import functools

import jax
import jax.numpy as jnp
from jax import lax
from jax.experimental import pallas as pl
from jax.experimental.pallas import tpu as pltpu
from jax.experimental.pallas import tpu_sc as plsc

N = 10000
E = 640000
D = 128
NSC = 2
NTILE = 16
NW = NSC * NTILE
EPW = E // NW
CHUNK = 400
NCHUNK = EPW // CHUNK
CHUNK_S = 160
NCHUNK_S = EPW // CHUNK_S
NPAD = 10240
ROWS_PT = NPAD // NTILE

_f32 = jnp.float32

_SC_PARAMS = pltpu.CompilerParams()
if "needs_layout_passes" in pltpu.CompilerParams.__dataclass_fields__:
    _SC_PARAMS = dataclasses.replace(_SC_PARAMS, needs_layout_passes=False)


def _pad2(w, rows, cols):
    return jnp.zeros((rows, cols), _f32).at[: w.shape[0], : w.shape[1]].set(w)


def _pad_bias(b, cols):
    return jnp.zeros((1, cols), _f32).at[0, : b.shape[0]].set(b)


def _prep_body(nf, h, jet, w0j, w0ns, w0hs, w0nd, w0hd, be0, psrc, pdst, z):
    dot = functools.partial(jnp.dot, preferred_element_type=_f32)
    psrc[...] = (dot(jet[...], w0j[...]) + dot(nf[...], w0ns[...])
                 + dot(h[...], w0hs[...]) + be0[...])
    pdst[...] = dot(nf[...], w0nd[...]) + dot(h[...], w0hd[...])
    z[:, 0:64] = nf[...]
    z[:, 64:128] = h[...]


def _gather_sum_kernel(psrc_hbm, pdst_hbm, src_hbm, dst_hbm, o_hbm,
                       idxs, idxd, gs, gd):
    c = lax.axis_index("c")
    s = lax.axis_index("s")
    ebase = (c * NTILE + s) * EPW

    @pl.loop(0, NCHUNK)
    def _chunks(k):
        base = ebase + k * CHUNK
        pltpu.sync_copy(src_hbm.at[pl.ds(base, CHUNK)], idxs)
        pltpu.sync_copy(dst_hbm.at[pl.ds(base, CHUNK)], idxd)
        pltpu.sync_copy(psrc_hbm.at[idxs], gs)
        pltpu.sync_copy(pdst_hbm.at[idxd], gd)

        @plsc.parallel_loop(0, CHUNK, step=1, unroll=4)
        def _rows(r):
            for j in range(D // 16):
                sl = (pl.ds(r, 1), pl.ds(j * 16, 16))
                gs.at[sl][...] = gs.at[sl][...] + gd.at[sl][...]

        pltpu.sync_copy(gs, o_hbm.at[pl.ds(base, CHUNK)])


def _edge_mlp_body(x1_ref, w1, b1, w2, b2, w3, b3, w4, b4, w5, b5, o_ref):
    dot = functools.partial(jnp.dot, preferred_element_type=_f32)
    bf = jnp.bfloat16
    x = jnp.maximum(x1_ref[...], 0.0).astype(bf)
    x = jnp.maximum(dot(x, w1[...]) + b1[...], 0.0).astype(bf)
    x = jnp.maximum(dot(x, w2[...]) + b2[...], 0.0).astype(bf)
    x = jnp.maximum(dot(x, w3[...]) + b3[...], 0.0).astype(bf)
    x = jnp.maximum(dot(x, w4[...]) + b4[...], 0.0).astype(bf)
    x = dot(x, w5[...]) + b5[...]
    o_ref[...] = jax.nn.sigmoid(x[:, 0:1])


def _scatter_kernel(z_hbm, src_hbm, dst_hbm, w_hbm, o_hbm,
                    idxs, idxd, wbuf, zbuf, agg):
    c = lax.axis_index("c")
    s = lax.axis_index("s")

    @pl.loop(0, CHUNK_S)
    def _zero(r):
        for j in range(D // 16):
            zbuf.at[pl.ds(r, 1), pl.ds(j * 16, 16)][...] = jnp.zeros((1, 16), _f32)

    rbase = s * ROWS_PT
    for off in range(0, ROWS_PT - CHUNK_S + 1, CHUNK_S):
        pltpu.sync_copy(zbuf, agg.at[pl.ds(rbase + off, CHUNK_S)])
    rem = ROWS_PT % CHUNK_S
    if rem:
        pltpu.sync_copy(zbuf.at[pl.ds(0, rem)],
                        agg.at[pl.ds(rbase + ROWS_PT - rem, rem)])
    plsc.subcore_barrier()

    ebase = (c * NTILE + s) * EPW

    @pl.loop(0, NCHUNK_S)
    def _chunks(k):
        base = ebase + k * CHUNK_S
        pltpu.sync_copy(src_hbm.at[pl.ds(base, CHUNK_S)], idxs)
        pltpu.sync_copy(dst_hbm.at[pl.ds(base, CHUNK_S)], idxd)
        pltpu.sync_copy(w_hbm.at[pl.ds(base, CHUNK_S)], wbuf)
        pltpu.sync_copy(z_hbm.at[idxs], zbuf)

        assert CHUNK_S % 16 == 0

        @plsc.parallel_loop(0, CHUNK_S, step=16)
        def _rows(r):
            wv = wbuf[pl.ds(r, 16)]
            for i in range(16):
                wi = wv[i]
                for j in range(D // 16):
                    sl = (pl.ds(r + i, 1), pl.ds(j * 16, 16))
                    zbuf.at[sl][...] = zbuf.at[sl][...] * wi

        pltpu.sync_copy(zbuf, agg.at[idxd], add=True)

    plsc.subcore_barrier()
    pltpu.sync_copy(agg.at[pl.ds(rbase, ROWS_PT)],
                    o_hbm.at[c, pl.ds(rbase, ROWS_PT)])


def _node_mlp_body(nf, h, jet, aggs, w0a, w0b, w0c, w0d,
                   w1, b1, w2, b2, w3, b3, w4, b4, o_ref):
    dot = functools.partial(jnp.dot, preferred_element_type=_f32)
    agg = aggs[0] + aggs[1]
    x = jnp.maximum(dot(nf[...], w0a[...]) + dot(h[...], w0b[...])
                    + dot(agg, w0c[...]) + dot(jet[...], w0d[...]), 0.0)
    x = jnp.maximum(dot(x, w1[...]) + b1[...], 0.0)
    x = jnp.maximum(dot(x, w2[...]) + b2[...], 0.0)
    x = jnp.maximum(dot(x, w3[...]) + b3[...], 0.0)
    o_ref[...] = dot(x, w4[...]) + b4[...]


def kernel(node_features, h, jet_features, edge_index,
           We0, be0, We1, be1, We2, be2, We3, be3, We4, be4, We5, be5,
           Wn0, Wn1, bn1, Wn2, bn2, Wn3, bn3, Wn4, bn4):
    src = edge_index[0]
    dst = edge_index[1]

    w0j = _pad2(We0[0:16], 16, D)
    w0ns = _pad2(We0[16:80], 64, D)
    w0hs = _pad2(We0[80:144], 64, D)
    w0nd = _pad2(We0[144:208], 64, D)
    w0hd = _pad2(We0[208:272], 64, D)
    be0p = _pad_bias(be0, D)
    w1 = _pad2(We1, 128, 128)
    b1 = _pad_bias(be1, 128)
    w2 = _pad2(We2, 128, 64)
    b2 = _pad_bias(be2, 64)
    w3 = _pad2(We3, 64, 32)
    b3 = _pad_bias(be3, 32)
    w4 = _pad2(We4, 32, 16)
    b4 = _pad_bias(be4, 16)
    w5 = _pad2(We5, 16, 8)
    b5 = _pad_bias(be5, 8)

    wn0a = _pad2(Wn0[0:64], 64, 512)
    wn0b = _pad2(Wn0[64:128], 64, 512)
    wn0c = _pad2(Wn0[128:256], 128, 512)
    wn0d = _pad2(Wn0[256:272], 16, 512)
    wn1 = _pad2(Wn1, 512, 512)
    bn1p = _pad_bias(bn1, 512)
    wn2 = _pad2(Wn2, 512, 512)
    bn2p = _pad_bias(bn2, 512)
    wn3 = _pad2(Wn3, 512, 256)
    bn3p = _pad_bias(bn3, 256)
    wn4 = _pad2(Wn4, 256, 64)
    bn4p = _pad_bias(bn4, 64)

    psrc, pdst, z = pl.pallas_call(
        _prep_body,
        out_shape=[jax.ShapeDtypeStruct((N, D), _f32)] * 3,
    )(node_features, h, jet_features, w0j, w0ns, w0hs, w0nd, w0hd, be0p)

    mesh = plsc.VectorSubcoreMesh(core_axis_name="c", subcore_axis_name="s")
    x1 = pl.kernel(
        _gather_sum_kernel,
        out_type=jax.ShapeDtypeStruct((E, D), _f32),
        mesh=mesh,
        scratch_types=[
            pltpu.VMEM((CHUNK,), jnp.int32),
            pltpu.VMEM((CHUNK,), jnp.int32),
            pltpu.VMEM((CHUNK, D), _f32),
            pltpu.VMEM((CHUNK, D), _f32),
        ],
    )(psrc, pdst, src, dst)

    MB = 8000
    e_weight = pl.pallas_call(
        _edge_mlp_body,
        grid=(E // MB,),
        in_specs=[
            pl.BlockSpec((MB, D), lambda i: (i, 0)),
            pl.BlockSpec((128, 128), lambda i: (0, 0)),
            pl.BlockSpec((1, 128), lambda i: (0, 0)),
            pl.BlockSpec((128, 64), lambda i: (0, 0)),
            pl.BlockSpec((1, 64), lambda i: (0, 0)),
            pl.BlockSpec((64, 32), lambda i: (0, 0)),
            pl.BlockSpec((1, 32), lambda i: (0, 0)),
            pl.BlockSpec((32, 16), lambda i: (0, 0)),
            pl.BlockSpec((1, 16), lambda i: (0, 0)),
            pl.BlockSpec((16, 8), lambda i: (0, 0)),
            pl.BlockSpec((1, 8), lambda i: (0, 0)),
        ],
        out_specs=pl.BlockSpec((MB, 1), lambda i: (i, 0)),
        out_shape=jax.ShapeDtypeStruct((E, 1), _f32),
    )(x1, w1.astype(jnp.bfloat16), b1, w2.astype(jnp.bfloat16), b2,
      w3.astype(jnp.bfloat16), b3, w4.astype(jnp.bfloat16), b4,
      w5.astype(jnp.bfloat16), b5)

    aggs = pl.kernel(
        _scatter_kernel,
        out_type=jax.ShapeDtypeStruct((NSC, NPAD, D), _f32),
        mesh=mesh,
        scratch_types=[
            pltpu.VMEM((CHUNK_S,), jnp.int32),
            pltpu.VMEM((CHUNK_S,), jnp.int32),
            pltpu.VMEM((CHUNK_S,), _f32),
            pltpu.VMEM((CHUNK_S, D), _f32),
            pltpu.VMEM_SHARED((NPAD, D), _f32),
        ],
    )(z, src, dst, e_weight.reshape(E))

    NB = 2000
    new_h = pl.pallas_call(
        _node_mlp_body,
        grid=(N // NB,),
        in_specs=[
            pl.BlockSpec((NB, 64), lambda i: (i, 0)),
            pl.BlockSpec((NB, 64), lambda i: (i, 0)),
            pl.BlockSpec((NB, 16), lambda i: (i, 0)),
            pl.BlockSpec((NSC, NB, 128), lambda i: (0, i, 0)),
            pl.BlockSpec((64, 512), lambda i: (0, 0)),
            pl.BlockSpec((64, 512), lambda i: (0, 0)),
            pl.BlockSpec((128, 512), lambda i: (0, 0)),
            pl.BlockSpec((16, 512), lambda i: (0, 0)),
            pl.BlockSpec((512, 512), lambda i: (0, 0)),
            pl.BlockSpec((1, 512), lambda i: (0, 0)),
            pl.BlockSpec((512, 512), lambda i: (0, 0)),
            pl.BlockSpec((1, 512), lambda i: (0, 0)),
            pl.BlockSpec((512, 256), lambda i: (0, 0)),
            pl.BlockSpec((1, 256), lambda i: (0, 0)),
            pl.BlockSpec((256, 64), lambda i: (0, 0)),
            pl.BlockSpec((1, 64), lambda i: (0, 0)),
        ],
        out_specs=pl.BlockSpec((NB, 64), lambda i: (i, 0)),
        out_shape=jax.ShapeDtypeStruct((N, 64), _f32),
    )(node_features, h, jet_features, aggs,
      wn0a, wn0b, wn0c, wn0d, wn1, bn1p, wn2, bn2p, wn3, bn3p, wn4, bn4p)

    return new_h, e_weight


# speedup vs baseline: 1.2159x; 1.2159x over previous
import dataclasses
import functools

import jax
import jax.numpy as jnp
from jax import lax
from jax.experimental import pallas as pl
from jax.experimental.pallas import tpu as pltpu
from jax.experimental.pallas import tpu_sc as plsc

N = 10000
E = 640000
D = 128
NSC = 2
NTILE = 16
NW = NSC * NTILE
EPW = E // NW
CHUNK = 200
SUP = 2000
NSUP = EPW // SUP
NCHP = SUP // CHUNK
CHUNK_S = 160
SUP_S = 800
NSUP_S = EPW // SUP_S
NCHP_S = SUP_S // CHUNK_S
NPAD = 10240
ROWS_PT = NPAD // NTILE

_f32 = jnp.float32

_SC_PARAMS = pltpu.CompilerParams()
if "needs_layout_passes" in pltpu.CompilerParams.__dataclass_fields__:
    _SC_PARAMS = dataclasses.replace(_SC_PARAMS, needs_layout_passes=False)


def _pad2(w, rows, cols):
    return jnp.zeros((rows, cols), _f32).at[: w.shape[0], : w.shape[1]].set(w)


def _pad_bias(b, cols):
    return jnp.zeros((1, cols), _f32).at[0, : b.shape[0]].set(b)


def _prep_body(nf, h, jet, w0j, w0ns, w0hs, w0nd, w0hd, be0, psrc, pdst, z):
    dot = functools.partial(jnp.dot, preferred_element_type=_f32)
    psrc[...] = (dot(jet[...], w0j[...]) + dot(nf[...], w0ns[...])
                 + dot(h[...], w0hs[...]) + be0[...])
    pdst[...] = dot(nf[...], w0nd[...]) + dot(h[...], w0hd[...])
    z[:, 0:64] = nf[...]
    z[:, 64:128] = h[...]


def _gather_sum_kernel(psrc_hbm, pdst_hbm, src_hbm, dst_hbm, o_hbm,
                       idxs_all, idxd_all, gs, gd, ob,
                       sg1, sg2, sout):
    c = lax.axis_index("c")
    s = lax.axis_index("s")
    ebase = (c * NTILE + s) * EPW

    @pl.loop(0, NSUP)
    def _sup(sp):
        sbase = ebase + sp * SUP
        i1 = pltpu.async_copy(src_hbm.at[pl.ds(sbase, SUP)], idxs_all, sg1)
        i2 = pltpu.async_copy(dst_hbm.at[pl.ds(sbase, SUP)], idxd_all, sg2)
        i1.wait()
        i2.wait()

        @pl.loop(0, NCHP)
        def _ck(k):
            off = k * CHUNK
            g1 = pltpu.async_copy(psrc_hbm.at[idxs_all.at[pl.ds(off, CHUNK)]],
                                  gs, sg1)
            g2 = pltpu.async_copy(pdst_hbm.at[idxd_all.at[pl.ds(off, CHUNK)]],
                                  gd, sg2)
            g1.wait()
            g2.wait()

            @pl.when(sp * NCHP + k >= 1)
            def _():
                pltpu.make_async_copy(ob, o_hbm.at[pl.ds(0, CHUNK)], sout).wait()

            @plsc.parallel_loop(0, CHUNK, step=1, unroll=4)
            def _rows(r):
                for j in range(D // 16):
                    sl = (pl.ds(r, 1), pl.ds(j * 16, 16))
                    ob.at[sl][...] = gs.at[sl][...] + gd.at[sl][...]

            pltpu.async_copy(ob, o_hbm.at[pl.ds(sbase + off, CHUNK)], sout)

    pltpu.make_async_copy(ob, o_hbm.at[pl.ds(0, CHUNK)], sout).wait()


def _edge_mlp_body(x1_ref, w1, b1, w2, b2, w3, b3, w4, b4, w5, b5, o_ref):
    dot = functools.partial(jnp.dot, preferred_element_type=_f32)
    bf = jnp.bfloat16
    x = jnp.maximum(x1_ref[...], 0.0).astype(bf)
    x = jnp.maximum(dot(x, w1[...]) + b1[...], 0.0).astype(bf)
    x = jnp.maximum(dot(x, w2[...]) + b2[...], 0.0).astype(bf)
    x = jnp.maximum(dot(x, w3[...]) + b3[...], 0.0).astype(bf)
    x = jnp.maximum(dot(x, w4[...]) + b4[...], 0.0).astype(bf)
    x = dot(x, w5[...]) + b5[...]
    o_ref[...] = jax.nn.sigmoid(x[:, 0:1])


def _scatter_kernel(z_hbm, src_hbm, dst_hbm, w_hbm, o_hbm,
                    idxs_all, w_all, idxd, zbuf, agg, s1, s2):
    c = lax.axis_index("c")
    s = lax.axis_index("s")

    @pl.loop(0, CHUNK_S)
    def _zero(r):
        for j in range(D // 16):
            zbuf.at[pl.ds(r, 1), pl.ds(j * 16, 16)][...] = jnp.zeros((1, 16), _f32)

    rbase = s * ROWS_PT
    assert ROWS_PT % CHUNK_S == 0
    for off in range(0, ROWS_PT, CHUNK_S):
        pltpu.sync_copy(zbuf, agg.at[pl.ds(rbase + off, CHUNK_S)])
    plsc.subcore_barrier()

    ebase = (c * NTILE + s) * EPW

    @pl.loop(0, NSUP_S)
    def _sup(sp):
        sbase = ebase + sp * SUP_S
        i1 = pltpu.async_copy(src_hbm.at[pl.ds(sbase, SUP_S)], idxs_all, s1)
        i2 = pltpu.async_copy(w_hbm.at[pl.ds(sbase, SUP_S)], w_all, s2)
        i1.wait()
        i2.wait()

        @pl.loop(0, NCHP_S)
        def _ck(k):
            off = k * CHUNK_S
            g1 = pltpu.async_copy(z_hbm.at[idxs_all.at[pl.ds(off, CHUNK_S)]],
                                  zbuf, s1)
            g2 = pltpu.async_copy(dst_hbm.at[pl.ds(sbase + off, CHUNK_S)],
                                  idxd, s2)
            g1.wait()
            g2.wait()
            assert CHUNK_S % 16 == 0

            @plsc.parallel_loop(0, CHUNK_S, step=16)
            def _rows(r):
                wv = w_all[pl.ds(off + r, 16)]
                for i in range(16):
                    wi = wv[i]
                    for j in range(D // 16):
                        sl = (pl.ds(r + i, 1), pl.ds(j * 16, 16))
                        zbuf.at[sl][...] = zbuf.at[sl][...] * wi

            pltpu.sync_copy(zbuf, agg.at[idxd], add=True)

    plsc.subcore_barrier()
    pltpu.sync_copy(agg.at[pl.ds(rbase, ROWS_PT)],
                    o_hbm.at[c, pl.ds(rbase, ROWS_PT)])


def _node_mlp_body(nf, h, jet, aggs, w0a, w0b, w0c, w0d,
                   w1, b1, w2, b2, w3, b3, w4, b4, o_ref):
    dot = functools.partial(jnp.dot, preferred_element_type=_f32)
    agg = aggs[0] + aggs[1]
    x = jnp.maximum(dot(nf[...], w0a[...]) + dot(h[...], w0b[...])
                    + dot(agg, w0c[...]) + dot(jet[...], w0d[...]), 0.0)
    x = jnp.maximum(dot(x, w1[...]) + b1[...], 0.0)
    x = jnp.maximum(dot(x, w2[...]) + b2[...], 0.0)
    x = jnp.maximum(dot(x, w3[...]) + b3[...], 0.0)
    o_ref[...] = dot(x, w4[...]) + b4[...]


def kernel(node_features, h, jet_features, edge_index,
           We0, be0, We1, be1, We2, be2, We3, be3, We4, be4, We5, be5,
           Wn0, Wn1, bn1, Wn2, bn2, Wn3, bn3, Wn4, bn4):
    src = edge_index[0]
    dst = edge_index[1]

    w0j = _pad2(We0[0:16], 16, D)
    w0ns = _pad2(We0[16:80], 64, D)
    w0hs = _pad2(We0[80:144], 64, D)
    w0nd = _pad2(We0[144:208], 64, D)
    w0hd = _pad2(We0[208:272], 64, D)
    be0p = _pad_bias(be0, D)
    w1 = _pad2(We1, 128, 128)
    b1 = _pad_bias(be1, 128)
    w2 = _pad2(We2, 128, 64)
    b2 = _pad_bias(be2, 64)
    w3 = _pad2(We3, 64, 32)
    b3 = _pad_bias(be3, 32)
    w4 = _pad2(We4, 32, 16)
    b4 = _pad_bias(be4, 16)
    w5 = _pad2(We5, 16, 8)
    b5 = _pad_bias(be5, 8)

    wn0a = _pad2(Wn0[0:64], 64, 512)
    wn0b = _pad2(Wn0[64:128], 64, 512)
    wn0c = _pad2(Wn0[128:256], 128, 512)
    wn0d = _pad2(Wn0[256:272], 16, 512)
    wn1 = _pad2(Wn1, 512, 512)
    bn1p = _pad_bias(bn1, 512)
    wn2 = _pad2(Wn2, 512, 512)
    bn2p = _pad_bias(bn2, 512)
    wn3 = _pad2(Wn3, 512, 256)
    bn3p = _pad_bias(bn3, 256)
    wn4 = _pad2(Wn4, 256, 64)
    bn4p = _pad_bias(bn4, 64)

    psrc, pdst, z = pl.pallas_call(
        _prep_body,
        out_shape=[jax.ShapeDtypeStruct((N, D), _f32)] * 3,
    )(node_features, h, jet_features, w0j, w0ns, w0hs, w0nd, w0hd, be0p)

    mesh = plsc.VectorSubcoreMesh(core_axis_name="c", subcore_axis_name="s")
    x1 = pl.kernel(
        _gather_sum_kernel,
        out_type=jax.ShapeDtypeStruct((E, D), _f32),
        mesh=mesh,
        scratch_types=[
            pltpu.VMEM((SUP,), jnp.int32),
            pltpu.VMEM((SUP,), jnp.int32),
            pltpu.VMEM((CHUNK, D), _f32),
            pltpu.VMEM((CHUNK, D), _f32),
            pltpu.VMEM((CHUNK, D), _f32),
            pltpu.SemaphoreType.DMA,
            pltpu.SemaphoreType.DMA,
            pltpu.SemaphoreType.DMA,
        ],
    )(psrc, pdst, src, dst)

    MB = 8000
    e_weight = pl.pallas_call(
        _edge_mlp_body,
        grid=(E // MB,),
        in_specs=[
            pl.BlockSpec((MB, D), lambda i: (i, 0)),
            pl.BlockSpec((128, 128), lambda i: (0, 0)),
            pl.BlockSpec((1, 128), lambda i: (0, 0)),
            pl.BlockSpec((128, 64), lambda i: (0, 0)),
            pl.BlockSpec((1, 64), lambda i: (0, 0)),
            pl.BlockSpec((64, 32), lambda i: (0, 0)),
            pl.BlockSpec((1, 32), lambda i: (0, 0)),
            pl.BlockSpec((32, 16), lambda i: (0, 0)),
            pl.BlockSpec((1, 16), lambda i: (0, 0)),
            pl.BlockSpec((16, 8), lambda i: (0, 0)),
            pl.BlockSpec((1, 8), lambda i: (0, 0)),
        ],
        out_specs=pl.BlockSpec((MB, 1), lambda i: (i, 0)),
        out_shape=jax.ShapeDtypeStruct((E, 1), _f32),
    )(x1, w1.astype(jnp.bfloat16), b1, w2.astype(jnp.bfloat16), b2,
      w3.astype(jnp.bfloat16), b3, w4.astype(jnp.bfloat16), b4,
      w5.astype(jnp.bfloat16), b5)

    aggs = pl.kernel(
        _scatter_kernel,
        out_type=jax.ShapeDtypeStruct((NSC, NPAD, D), _f32),
        mesh=mesh,
        scratch_types=[
            pltpu.VMEM((SUP_S,), jnp.int32),
            pltpu.VMEM((SUP_S,), _f32),
            pltpu.VMEM((CHUNK_S,), jnp.int32),
            pltpu.VMEM((CHUNK_S, D), _f32),
            pltpu.VMEM_SHARED((NPAD, D), _f32),
            pltpu.SemaphoreType.DMA,
            pltpu.SemaphoreType.DMA,
        ],
    )(z, src, dst, e_weight.reshape(E))

    NB = 2000
    new_h = pl.pallas_call(
        _node_mlp_body,
        grid=(N // NB,),
        in_specs=[
            pl.BlockSpec((NB, 64), lambda i: (i, 0)),
            pl.BlockSpec((NB, 64), lambda i: (i, 0)),
            pl.BlockSpec((NB, 16), lambda i: (i, 0)),
            pl.BlockSpec((NSC, NB, 128), lambda i: (0, i, 0)),
            pl.BlockSpec((64, 512), lambda i: (0, 0)),
            pl.BlockSpec((64, 512), lambda i: (0, 0)),
            pl.BlockSpec((128, 512), lambda i: (0, 0)),
            pl.BlockSpec((16, 512), lambda i: (0, 0)),
            pl.BlockSpec((512, 512), lambda i: (0, 0)),
            pl.BlockSpec((1, 512), lambda i: (0, 0)),
            pl.BlockSpec((512, 512), lambda i: (0, 0)),
            pl.BlockSpec((1, 512), lambda i: (0, 0)),
            pl.BlockSpec((512, 256), lambda i: (0, 0)),
            pl.BlockSpec((1, 256), lambda i: (0, 0)),
            pl.BlockSpec((256, 64), lambda i: (0, 0)),
            pl.BlockSpec((1, 64), lambda i: (0, 0)),
        ],
        out_specs=pl.BlockSpec((NB, 64), lambda i: (i, 0)),
        out_shape=jax.ShapeDtypeStruct((N, 64), _f32),
    )(node_features, h, jet_features, aggs,
      wn0a, wn0b, wn0c, wn0d, wn1, bn1p, wn2, bn2p, wn3, bn3p, wn4, bn4p)

    return new_h, e_weight


# speedup vs baseline: 1.4616x; 1.2021x over previous
import dataclasses
import functools

import jax
import jax.numpy as jnp
from jax import lax
from jax.experimental import pallas as pl
from jax.experimental.pallas import tpu as pltpu
from jax.experimental.pallas import tpu_sc as plsc

N = 10000
E = 640000
D = 128
NSC = 2
NTILE = 16
NW = NSC * NTILE
STRIPES = 2
EH = E // STRIPES
EPW = EH // NW
CHUNK = 200
SUP = 2000
NSUP = EPW // SUP
NCHP = SUP // CHUNK
CHUNK_S = 200
SUP_S = 1000
NSUP_S = EPW // SUP_S
NCHP_S = SUP_S // CHUNK_S
NPAD = 10240
ROWS_PT = NPAD // NTILE

_f32 = jnp.float32

_SC_PARAMS = pltpu.CompilerParams()
if "needs_layout_passes" in pltpu.CompilerParams.__dataclass_fields__:
    _SC_PARAMS = dataclasses.replace(_SC_PARAMS, needs_layout_passes=False)


def _pad2(w, rows, cols):
    return jnp.zeros((rows, cols), _f32).at[: w.shape[0], : w.shape[1]].set(w)


def _pad_bias(b, cols):
    return jnp.zeros((1, cols), _f32).at[0, : b.shape[0]].set(b)


def _prep_body(nf, h, jet, w0j, w0ns, w0hs, w0nd, w0hd, be0, psrc, pdst, z):
    dot = functools.partial(jnp.dot, preferred_element_type=_f32)
    psrc[...] = (dot(jet[...], w0j[...]) + dot(nf[...], w0ns[...])
                 + dot(h[...], w0hs[...]) + be0[...])
    pdst[...] = dot(nf[...], w0nd[...]) + dot(h[...], w0hd[...])
    z[:, 0:64] = nf[...]
    z[:, 64:128] = h[...]


def _gather_sum_kernel(psrc_hbm, pdst_hbm, src_hbm, dst_hbm, o_hbm,
                       idxs_all, idxd_all, gs, gd, ob,
                       sg1, sg2, sout):
    c = lax.axis_index("c")
    s = lax.axis_index("s")
    ebase = (c * NTILE + s) * EPW

    @pl.loop(0, NSUP)
    def _sup(sp):
        sbase = ebase + sp * SUP
        i1 = pltpu.async_copy(src_hbm.at[pl.ds(sbase, SUP)], idxs_all, sg1)
        i2 = pltpu.async_copy(dst_hbm.at[pl.ds(sbase, SUP)], idxd_all, sg2)
        i1.wait()
        i2.wait()

        @pl.loop(0, NCHP)
        def _ck(k):
            off = k * CHUNK
            g1 = pltpu.async_copy(psrc_hbm.at[idxs_all.at[pl.ds(off, CHUNK)]],
                                  gs, sg1)
            g2 = pltpu.async_copy(pdst_hbm.at[idxd_all.at[pl.ds(off, CHUNK)]],
                                  gd, sg2)
            g1.wait()
            g2.wait()

            @pl.when(sp * NCHP + k >= 1)
            def _():
                pltpu.make_async_copy(ob, o_hbm.at[pl.ds(0, CHUNK)], sout).wait()

            @plsc.parallel_loop(0, CHUNK, step=1, unroll=4)
            def _rows(r):
                for j in range(D // 16):
                    sl = (pl.ds(r, 1), pl.ds(j * 16, 16))
                    ob.at[sl][...] = gs.at[sl][...] + gd.at[sl][...]

            pltpu.async_copy(ob, o_hbm.at[pl.ds(sbase + off, CHUNK)], sout)

    pltpu.make_async_copy(ob, o_hbm.at[pl.ds(0, CHUNK)], sout).wait()


def _edge_mlp_body(x1_ref, w1, b1, w2, b2, w3, b3, w4, b4, w5, b5, o_ref):
    dot = functools.partial(jnp.dot, preferred_element_type=_f32)
    bf = jnp.bfloat16
    x = jnp.maximum(x1_ref[...], 0.0).astype(bf)
    x = jnp.maximum(dot(x, w1[...]) + b1[...], 0.0).astype(bf)
    x = jnp.maximum(dot(x, w2[...]) + b2[...], 0.0).astype(bf)
    x = jnp.maximum(dot(x, w3[...]) + b3[...], 0.0).astype(bf)
    x = jnp.maximum(dot(x, w4[...]) + b4[...], 0.0).astype(bf)
    x = dot(x, w5[...]) + b5[...]
    o_ref[...] = jax.nn.sigmoid(x[:, 0:1])


def _scatter_kernel(z_hbm, src_hbm, dst_hbm, w_hbm, o_hbm,
                    idxs_all, w_all, idxd, zbuf, agg, s1, s2):
    c = lax.axis_index("c")
    s = lax.axis_index("s")

    @pl.loop(0, CHUNK_S)
    def _zero(r):
        for j in range(D // 16):
            zbuf.at[pl.ds(r, 1), pl.ds(j * 16, 16)][...] = jnp.zeros((1, 16), _f32)

    rbase = s * ROWS_PT
    for off in range(0, ROWS_PT - CHUNK_S + 1, CHUNK_S):
        pltpu.sync_copy(zbuf, agg.at[pl.ds(rbase + off, CHUNK_S)])
    _rem = ROWS_PT % CHUNK_S
    if _rem:
        pltpu.sync_copy(zbuf.at[pl.ds(0, _rem)],
                        agg.at[pl.ds(rbase + ROWS_PT - _rem, _rem)])
    plsc.subcore_barrier()

    ebase = (c * NTILE + s) * EPW

    @pl.loop(0, NSUP_S)
    def _sup(sp):
        sbase = ebase + sp * SUP_S
        i1 = pltpu.async_copy(src_hbm.at[pl.ds(sbase, SUP_S)], idxs_all, s1)
        i2 = pltpu.async_copy(w_hbm.at[pl.ds(sbase, SUP_S)], w_all, s2)
        i1.wait()
        i2.wait()

        @pl.loop(0, NCHP_S)
        def _ck(k):
            off = k * CHUNK_S
            g1 = pltpu.async_copy(z_hbm.at[idxs_all.at[pl.ds(off, CHUNK_S)]],
                                  zbuf, s1)
            g2 = pltpu.async_copy(dst_hbm.at[pl.ds(sbase + off, CHUNK_S)],
                                  idxd, s2)
            g1.wait()
            g2.wait()
            nmain = (CHUNK_S // 16) * 16

            @plsc.parallel_loop(0, nmain, step=16)
            def _rows(r):
                wv = w_all[pl.ds(off + r, 16)]
                for i in range(16):
                    wi = wv[i]
                    for j in range(D // 16):
                        sl = (pl.ds(r + i, 1), pl.ds(j * 16, 16))
                        zbuf.at[sl][...] = zbuf.at[sl][...] * wi

            if CHUNK_S > nmain:
                wv = w_all[pl.ds(off + CHUNK_S - 16, 16)]
                for i in range(16 - (CHUNK_S - nmain), 16):
                    wi = wv[i]
                    r = CHUNK_S - 16 + i
                    for j in range(D // 16):
                        sl = (pl.ds(r, 1), pl.ds(j * 16, 16))
                        zbuf.at[sl][...] = zbuf.at[sl][...] * wi

            pltpu.sync_copy(zbuf, agg.at[idxd], add=True)

    plsc.subcore_barrier()
    pltpu.sync_copy(agg.at[pl.ds(rbase, ROWS_PT)],
                    o_hbm.at[c, pl.ds(rbase, ROWS_PT)])


def _node_mlp_body(nf, h, jet, aggs_a, aggs_b, w0a, w0b, w0c, w0d,
                   w1, b1, w2, b2, w3, b3, w4, b4, o_ref):
    dot = functools.partial(jnp.dot, preferred_element_type=_f32)
    agg = (aggs_a[0] + aggs_a[1]) + (aggs_b[0] + aggs_b[1])
    x = jnp.maximum(dot(nf[...], w0a[...]) + dot(h[...], w0b[...])
                    + dot(agg, w0c[...]) + dot(jet[...], w0d[...]), 0.0)
    x = jnp.maximum(dot(x, w1[...]) + b1[...], 0.0)
    x = jnp.maximum(dot(x, w2[...]) + b2[...], 0.0)
    x = jnp.maximum(dot(x, w3[...]) + b3[...], 0.0)
    o_ref[...] = dot(x, w4[...]) + b4[...]


def kernel(node_features, h, jet_features, edge_index,
           We0, be0, We1, be1, We2, be2, We3, be3, We4, be4, We5, be5,
           Wn0, Wn1, bn1, Wn2, bn2, Wn3, bn3, Wn4, bn4):
    src = edge_index[0]
    dst = edge_index[1]

    w0j = _pad2(We0[0:16], 16, D)
    w0ns = _pad2(We0[16:80], 64, D)
    w0hs = _pad2(We0[80:144], 64, D)
    w0nd = _pad2(We0[144:208], 64, D)
    w0hd = _pad2(We0[208:272], 64, D)
    be0p = _pad_bias(be0, D)
    w1 = _pad2(We1, 128, 128)
    b1 = _pad_bias(be1, 128)
    w2 = _pad2(We2, 128, 64)
    b2 = _pad_bias(be2, 64)
    w3 = _pad2(We3, 64, 32)
    b3 = _pad_bias(be3, 32)
    w4 = _pad2(We4, 32, 16)
    b4 = _pad_bias(be4, 16)
    w5 = _pad2(We5, 16, 8)
    b5 = _pad_bias(be5, 8)

    wn0a = _pad2(Wn0[0:64], 64, 512)
    wn0b = _pad2(Wn0[64:128], 64, 512)
    wn0c = _pad2(Wn0[128:256], 128, 512)
    wn0d = _pad2(Wn0[256:272], 16, 512)
    wn1 = _pad2(Wn1, 512, 512)
    bn1p = _pad_bias(bn1, 512)
    wn2 = _pad2(Wn2, 512, 512)
    bn2p = _pad_bias(bn2, 512)
    wn3 = _pad2(Wn3, 512, 256)
    bn3p = _pad_bias(bn3, 256)
    wn4 = _pad2(Wn4, 256, 64)
    bn4p = _pad_bias(bn4, 64)

    psrc, pdst, z = pl.pallas_call(
        _prep_body,
        out_shape=[jax.ShapeDtypeStruct((N, D), _f32)] * 3,
    )(node_features, h, jet_features, w0j, w0ns, w0hs, w0nd, w0hd, be0p)

    mesh = plsc.VectorSubcoreMesh(core_axis_name="c", subcore_axis_name="s")
    MB = 8000

    def gather_sum(src_h, dst_h):
        return pl.kernel(
            _gather_sum_kernel,
            out_type=jax.ShapeDtypeStruct((EH, D), _f32),
            mesh=mesh,
            scratch_types=[
                pltpu.VMEM((SUP,), jnp.int32),
                pltpu.VMEM((SUP,), jnp.int32),
                pltpu.VMEM((CHUNK, D), _f32),
                pltpu.VMEM((CHUNK, D), _f32),
                pltpu.VMEM((CHUNK, D), _f32),
                pltpu.SemaphoreType.DMA,
                pltpu.SemaphoreType.DMA,
                pltpu.SemaphoreType.DMA,
            ],
        )(psrc, pdst, src_h, dst_h)

    def edge_mlp(x1_h):
        return pl.pallas_call(
            _edge_mlp_body,
            grid=(EH // MB,),
            in_specs=[
                pl.BlockSpec((MB, D), lambda i: (i, 0)),
                pl.BlockSpec((128, 128), lambda i: (0, 0)),
                pl.BlockSpec((1, 128), lambda i: (0, 0)),
                pl.BlockSpec((128, 64), lambda i: (0, 0)),
                pl.BlockSpec((1, 64), lambda i: (0, 0)),
                pl.BlockSpec((64, 32), lambda i: (0, 0)),
                pl.BlockSpec((1, 32), lambda i: (0, 0)),
                pl.BlockSpec((32, 16), lambda i: (0, 0)),
                pl.BlockSpec((1, 16), lambda i: (0, 0)),
                pl.BlockSpec((16, 8), lambda i: (0, 0)),
                pl.BlockSpec((1, 8), lambda i: (0, 0)),
            ],
            out_specs=pl.BlockSpec((MB, 1), lambda i: (i, 0)),
            out_shape=jax.ShapeDtypeStruct((EH, 1), _f32),
        )(x1_h, w1.astype(jnp.bfloat16), b1, w2.astype(jnp.bfloat16), b2,
          w3.astype(jnp.bfloat16), b3, w4.astype(jnp.bfloat16), b4,
          w5.astype(jnp.bfloat16), b5)

    def scatter(src_h, dst_h, w_h):
        return pl.kernel(
            _scatter_kernel,
            out_type=jax.ShapeDtypeStruct((NSC, NPAD, D), _f32),
            mesh=mesh,
            scratch_types=[
                pltpu.VMEM((SUP_S,), jnp.int32),
                pltpu.VMEM((SUP_S,), _f32),
                pltpu.VMEM((CHUNK_S,), jnp.int32),
                pltpu.VMEM((CHUNK_S, D), _f32),
                pltpu.VMEM_SHARED((NPAD, D), _f32),
                pltpu.SemaphoreType.DMA,
                pltpu.SemaphoreType.DMA,
            ],
        )(z, src_h, dst_h, w_h.reshape(EH))

    src_s = [src[i * EH:(i + 1) * EH] for i in range(STRIPES)]
    dst_s = [dst[i * EH:(i + 1) * EH] for i in range(STRIPES)]
    x1_s = [gather_sum(src_s[i], dst_s[i]) for i in range(STRIPES)]
    ew_s = [edge_mlp(x1_s[i]) for i in range(STRIPES)]
    agg_s = [scatter(src_s[i], dst_s[i], ew_s[i]) for i in range(STRIPES)]
    e_weight = jnp.concatenate(ew_s, axis=0)

    NB = 2000
    new_h = pl.pallas_call(
        _node_mlp_body,
        grid=(N // NB,),
        in_specs=[
            pl.BlockSpec((NB, 64), lambda i: (i, 0)),
            pl.BlockSpec((NB, 64), lambda i: (i, 0)),
            pl.BlockSpec((NB, 16), lambda i: (i, 0)),
            pl.BlockSpec((NSC, NB, 128), lambda i: (0, i, 0)),
            pl.BlockSpec((NSC, NB, 128), lambda i: (0, i, 0)),
            pl.BlockSpec((64, 512), lambda i: (0, 0)),
            pl.BlockSpec((64, 512), lambda i: (0, 0)),
            pl.BlockSpec((128, 512), lambda i: (0, 0)),
            pl.BlockSpec((16, 512), lambda i: (0, 0)),
            pl.BlockSpec((512, 512), lambda i: (0, 0)),
            pl.BlockSpec((1, 512), lambda i: (0, 0)),
            pl.BlockSpec((512, 512), lambda i: (0, 0)),
            pl.BlockSpec((1, 512), lambda i: (0, 0)),
            pl.BlockSpec((512, 256), lambda i: (0, 0)),
            pl.BlockSpec((1, 256), lambda i: (0, 0)),
            pl.BlockSpec((256, 64), lambda i: (0, 0)),
            pl.BlockSpec((1, 64), lambda i: (0, 0)),
        ],
        out_specs=pl.BlockSpec((NB, 64), lambda i: (i, 0)),
        out_shape=jax.ShapeDtypeStruct((N, 64), _f32),
    )(node_features, h, jet_features, agg_s[0], agg_s[1],
      wn0a, wn0b, wn0c, wn0d, wn1, bn1p, wn2, bn2p, wn3, bn3p, wn4, bn4p)

    return new_h, e_weight
